# Initial kernel scaffold; baseline (speedup 1.0000x reference)
import functools

import jax
import jax.numpy as jnp
from jax import lax
from jax.experimental import pallas as pl
from jax.experimental.pallas import tpu as pltpu
from jax.experimental.pallas import tpu_sc as plsc

N = 10000
E = 320000
D = 128

NC = 2
NS = 16
NW = NC * NS
EB = 128
NBLK = E // EB
ROWS_PER_TILE = N // NS
ZR = 125

_sc_mesh = plsc.VectorSubcoreMesh(
    core_axis_name="c", subcore_axis_name="s", num_cores=NC, num_subcores=NS)


@functools.partial(
    pl.kernel,
    out_type=jax.ShapeDtypeStruct((2 * N, D), jnp.float32),
    mesh=_sc_mesh,
    scratch_types=[
        pltpu.VMEM_SHARED((N, D), jnp.float32),
        pltpu.VMEM((EB,), jnp.int32),
        pltpu.VMEM((EB,), jnp.int32),
        pltpu.VMEM((EB, D), jnp.float32),
        pltpu.VMEM((ZR, D), jnp.float32),
        pltpu.SemaphoreType.DMA,
    ],
)
def _segsum_sc(x_hbm, src_hbm, dst_hbm, out_hbm, acc, idx_s, idx_d, rows,
               zbuf, sem):
    cid = lax.axis_index("c")
    sid = lax.axis_index("s")
    wid = sid * NC + cid

    def _zrow(k, carry):
        i = k // (D // 16)
        j = k - i * (D // 16)
        zbuf[i, pl.ds(j * 16, 16)] = jnp.zeros((16,), jnp.float32)
        return carry
    lax.fori_loop(0, ZR * (D // 16), _zrow, 0)
    for k in range(ROWS_PER_TILE // ZR):
        pltpu.sync_copy(zbuf, acc.at[pl.ds(sid * ROWS_PER_TILE + k * ZR, ZR)])
    plsc.subcore_barrier()

    base_blocks = NBLK // NW
    extra = NBLK - base_blocks * NW
    nblk_w = base_blocks + jnp.where(wid < extra, 1, 0)

    def _body(j, carry):
        off = (wid + j * NW) * EB
        pltpu.sync_copy(src_hbm.at[pl.ds(off, EB)], idx_s)
        pltpu.sync_copy(dst_hbm.at[pl.ds(off, EB)], idx_d)
        pltpu.async_copy(x_hbm.at[idx_s], rows, sem).wait()
        pltpu.sync_copy(rows, acc.at[idx_d], add=True)
        return carry
    lax.fori_loop(0, nblk_w, _body, 0)
    plsc.subcore_barrier()

    r0 = sid * ROWS_PER_TILE
    pltpu.sync_copy(acc.at[pl.ds(r0, ROWS_PER_TILE)],
                    out_hbm.at[pl.ds(cid * N + r0, ROWS_PER_TILE)])


def _mlp_body(with_post_bn, x_ref, p_ref, w1_ref, b1_ref, g1_ref, be1_ref,
              w2_ref, b2_ref, g3_ref, be3_ref, o_ref):
    u = x_ref[...] + p_ref[0:N, :] + p_ref[N:2 * N, :]
    t = jnp.dot(u, w1_ref[...], preferred_element_type=jnp.float32)
    t = t + b1_ref[...]
    mean = jnp.mean(t, axis=0, keepdims=True)
    var = jnp.mean((t - mean) ** 2, axis=0, keepdims=True)
    t = (t - mean) / jnp.sqrt(var + 1e-5) * g1_ref[...] + be1_ref[...]
    t = jnp.maximum(t, 0.0)
    t = jnp.dot(t, w2_ref[...], preferred_element_type=jnp.float32)
    t = t + b2_ref[...]
    if with_post_bn:
        mean = jnp.mean(t, axis=0, keepdims=True)
        var = jnp.mean((t - mean) ** 2, axis=0, keepdims=True)
        t = (t - mean) / jnp.sqrt(var + 1e-5) * g3_ref[...] + be3_ref[...]
        t = jnp.maximum(t, 0.0)
    o_ref[...] = t


def _mlp_tc(x, parts, w1, b1, g1, be1, w2, b2, g3, be3, with_post_bn):
    r2 = lambda v: v.reshape(1, D)
    return pl.pallas_call(
        functools.partial(_mlp_body, with_post_bn),
        out_shape=jax.ShapeDtypeStruct((N, D), jnp.float32),
    )(x, parts, w1, r2(b1), r2(g1), r2(be1), w2, r2(b2), r2(g3), r2(be3))


def kernel(x, edge_index, w0_1, b0_1, g0_1, be0_1, w0_2, b0_2, g0_3, be0_3,
           w1_1, b1_1, g1_1, be1_1, w1_2, b1_2):
    src = edge_index[0]
    dst = edge_index[1]
    parts0 = _segsum_sc(x, src, dst)
    h = _mlp_tc(x, parts0, w0_1, b0_1, g0_1, be0_1, w0_2, b0_2, g0_3, be0_3,
                with_post_bn=True)
    parts1 = _segsum_sc(h, src, dst)
    out = _mlp_tc(h, parts1, w1_1, b1_1, g1_1, be1_1, w1_2, b1_2, g1_1, be1_1,
                  with_post_bn=False)
    return out


# speedup vs baseline: 5.7557x; 5.7557x over previous
import functools

import jax
import jax.numpy as jnp
from jax import lax
from jax.experimental import pallas as pl
from jax.experimental.pallas import tpu as pltpu
from jax.experimental.pallas import tpu_sc as plsc

N = 10000
E = 320000
D = 128

NC = 2
NS = 16
NW = NC * NS
EB = 128
NBLK = E // EB
ROWS_PER_TILE = 624
REM_ROWS = N - NS * ROWS_PER_TILE
ZR = 104

_sc_mesh = plsc.VectorSubcoreMesh(
    core_axis_name="c", subcore_axis_name="s", num_cores=NC, num_subcores=NS)


@functools.partial(
    pl.kernel,
    out_type=jax.ShapeDtypeStruct((2 * N, D), jnp.float32),
    mesh=_sc_mesh,
    scratch_types=[
        pltpu.VMEM_SHARED((N, D), jnp.float32),
        pltpu.VMEM((EB,), jnp.int32),
        pltpu.VMEM((EB,), jnp.int32),
        pltpu.VMEM((EB, D), jnp.float32),
        pltpu.VMEM((ZR, D), jnp.float32),
        pltpu.SemaphoreType.DMA,
    ],
)
def _segsum_sc(x_hbm, src_hbm, dst_hbm, out_hbm, acc, idx_s, idx_d, rows,
               zbuf, sem):
    cid = lax.axis_index("c")
    sid = lax.axis_index("s")
    wid = sid * NC + cid

    def _zrow(k, carry):
        i = k // (D // 16)
        j = k - i * (D // 16)
        zbuf[i, pl.ds(j * 16, 16)] = jnp.zeros((16,), jnp.float32)
        return carry
    lax.fori_loop(0, ZR * (D // 16), _zrow, 0)
    for k in range(ROWS_PER_TILE // ZR):
        pltpu.sync_copy(zbuf, acc.at[pl.ds(sid * ROWS_PER_TILE + k * ZR, ZR)])

    @pl.when(sid == NS - 1)
    def _zero_rem():
        pltpu.sync_copy(zbuf.at[pl.ds(0, REM_ROWS)],
                        acc.at[pl.ds(NS * ROWS_PER_TILE, REM_ROWS)])
    plsc.subcore_barrier()

    base_blocks = NBLK // NW
    extra = NBLK - base_blocks * NW
    nblk_w = base_blocks + jnp.where(wid < extra, 1, 0)

    def _body(j, carry):
        off = (wid + j * NW) * EB
        pltpu.sync_copy(src_hbm.at[pl.ds(off, EB)], idx_s)
        pltpu.sync_copy(dst_hbm.at[pl.ds(off, EB)], idx_d)
        pltpu.async_copy(x_hbm.at[idx_s], rows, sem).wait()
        pltpu.sync_copy(rows, acc.at[idx_d], add=True)
        return carry
    lax.fori_loop(0, nblk_w, _body, 0)
    plsc.subcore_barrier()

    r0 = sid * ROWS_PER_TILE
    pltpu.sync_copy(acc.at[pl.ds(r0, ROWS_PER_TILE)],
                    out_hbm.at[pl.ds(cid * N + r0, ROWS_PER_TILE)])

    @pl.when(sid == NS - 1)
    def _pub_rem():
        pltpu.sync_copy(acc.at[pl.ds(NS * ROWS_PER_TILE, REM_ROWS)],
                        out_hbm.at[pl.ds(cid * N + NS * ROWS_PER_TILE,
                                         REM_ROWS)])


def _mlp_body(with_post_bn, x_ref, p_ref, w1_ref, b1_ref, g1_ref, be1_ref,
              w2_ref, b2_ref, g3_ref, be3_ref, o_ref):
    u = x_ref[...] + p_ref[0:N, :] + p_ref[N:2 * N, :]
    t = jnp.dot(u, w1_ref[...], preferred_element_type=jnp.float32)
    t = t + b1_ref[...]
    mean = jnp.mean(t, axis=0, keepdims=True)
    var = jnp.mean((t - mean) ** 2, axis=0, keepdims=True)
    t = (t - mean) / jnp.sqrt(var + 1e-5) * g1_ref[...] + be1_ref[...]
    t = jnp.maximum(t, 0.0)
    t = jnp.dot(t, w2_ref[...], preferred_element_type=jnp.float32)
    t = t + b2_ref[...]
    if with_post_bn:
        mean = jnp.mean(t, axis=0, keepdims=True)
        var = jnp.mean((t - mean) ** 2, axis=0, keepdims=True)
        t = (t - mean) / jnp.sqrt(var + 1e-5) * g3_ref[...] + be3_ref[...]
        t = jnp.maximum(t, 0.0)
    o_ref[...] = t


def _mlp_tc(x, parts, w1, b1, g1, be1, w2, b2, g3, be3, with_post_bn):
    r2 = lambda v: v.reshape(1, D)
    return pl.pallas_call(
        functools.partial(_mlp_body, with_post_bn),
        out_shape=jax.ShapeDtypeStruct((N, D), jnp.float32),
    )(x, parts, w1, r2(b1), r2(g1), r2(be1), w2, r2(b2), r2(g3), r2(be3))


def kernel(x, edge_index, w0_1, b0_1, g0_1, be0_1, w0_2, b0_2, g0_3, be0_3,
           w1_1, b1_1, g1_1, be1_1, w1_2, b1_2):
    src = edge_index[0]
    dst = edge_index[1]
    parts0 = _segsum_sc(x, src, dst)
    h = _mlp_tc(x, parts0, w0_1, b0_1, g0_1, be0_1, w0_2, b0_2, g0_3, be0_3,
                with_post_bn=True)
    parts1 = _segsum_sc(h, src, dst)
    out = _mlp_tc(h, parts1, w1_1, b1_1, g1_1, be1_1, w1_2, b1_2, g1_1, be1_1,
                  with_post_bn=False)
    return out
